# Initial kernel scaffold; baseline (speedup 1.0000x reference)
import jax
import jax.numpy as jnp
from jax.experimental import pallas as pl


def kernel(x, edge_index, batch, W1, b1, g1, be1, rm1, rv1, W2, b2, g2, be2, rm2, rv2, Wr1, br1, Wr2, br2):
    raise NotImplementedError("write your pallas kernel here")


# speedup vs baseline: 15.9421x; 15.9421x over previous
import functools

import jax
import jax.numpy as jnp
from jax import lax
from jax.experimental import pallas as pl
from jax.experimental.pallas import tpu as pltpu
from jax.experimental.pallas import tpu_sc as plsc

_NC, _NS, _L = 2, 16, 16
_NW = _NC * _NS

_N, _F, _H, _G = 10000, 128, 64, 64
_RH = max(_H // 2, 16)
_EPS = 1e-5
_NPAD = 10240
_CHUNK = 128
_DEGW = 16
_STRIPE = _NPAD // _NS
_PR = _NPAD // _NW
_TS = (_G + 1) * _H
_BR = 1024


def _mesh():
    return plsc.VectorSubcoreMesh(core_axis_name="c", subcore_axis_name="s")


def _make_deg(kc):
    @functools.partial(
        pl.kernel,
        out_type=jax.ShapeDtypeStruct((_NC, _NPAD, _DEGW), jnp.float32),
        mesh=_mesh(),
        compiler_params=pltpu.CompilerParams(use_tc_tiling_on_sc=False, needs_layout_passes=False),
        scratch_types=[
            pltpu.VMEM((kc, _CHUNK), jnp.int32),
            pltpu.VMEM((_CHUNK, _DEGW), jnp.float32),
            pltpu.VMEM_SHARED((_NPAD, _DEGW), jnp.float32),
        ],
    )
    def deg_kernel(dst_hbm, zeros_hbm, ones_hbm, out_hbm, dst_v, ones_v, deg_sh):
        cid = lax.axis_index("c")
        sid = lax.axis_index("s")
        wid = cid * _NS + sid
        pltpu.sync_copy(zeros_hbm.at[pl.ds(sid * _STRIPE, _STRIPE)],
                        deg_sh.at[pl.ds(sid * _STRIPE, _STRIPE)])
        pltpu.sync_copy(ones_hbm, ones_v)
        pltpu.sync_copy(dst_hbm.at[wid], dst_v)
        plsc.subcore_barrier()

        def step(j, c):
            pltpu.sync_copy(ones_v, deg_sh.at[dst_v.at[j]], add=True)
            return c

        lax.fori_loop(0, kc, step, 0)
        plsc.subcore_barrier()
        pltpu.sync_copy(deg_sh.at[pl.ds(sid * _STRIPE, _STRIPE)],
                        out_hbm.at[cid, pl.ds(sid * _STRIPE, _STRIPE)])

    return deg_kernel


def _make_agg(kc):
    @functools.partial(
        pl.kernel,
        out_type=jax.ShapeDtypeStruct((_NC, _NPAD, _H), jnp.float32),
        mesh=_mesh(),
        compiler_params=pltpu.CompilerParams(use_tc_tiling_on_sc=False, needs_layout_passes=False),
        scratch_types=[
            pltpu.VMEM((kc, _CHUNK), jnp.int32),
            pltpu.VMEM((kc, _CHUNK), jnp.int32),
            pltpu.VMEM((_CHUNK, _H), jnp.float32),
            pltpu.VMEM((_CHUNK, _H), jnp.float32),
            pltpu.VMEM_SHARED((_NPAD, _H), jnp.float32),
            pltpu.SemaphoreType.DMA,
            pltpu.SemaphoreType.DMA,
        ],
    )
    def agg_kernel(hs_hbm, src_hbm, dst_hbm, zeros_hbm, out_hbm,
                   src_v, dst_v, rows0, rows1, agg_sh, sem0, sem1):
        cid = lax.axis_index("c")
        sid = lax.axis_index("s")
        wid = cid * _NS + sid
        pltpu.sync_copy(zeros_hbm.at[pl.ds(sid * _STRIPE, _STRIPE)],
                        agg_sh.at[pl.ds(sid * _STRIPE, _STRIPE)])
        pltpu.sync_copy(src_hbm.at[wid], src_v)
        pltpu.sync_copy(dst_hbm.at[wid], dst_v)
        plsc.subcore_barrier()

        pltpu.async_copy(hs_hbm.at[src_v.at[0]], rows0, sem0)
        pltpu.async_copy(hs_hbm.at[src_v.at[1]], rows1, sem1)

        def step(p, c):
            j0 = 2 * p
            j1 = 2 * p + 1
            pltpu.make_async_copy(hs_hbm.at[src_v.at[j0]], rows0, sem0).wait()
            pltpu.sync_copy(rows0, agg_sh.at[dst_v.at[j0]], add=True)

            @pl.when(j0 + 2 < kc)
            def _():
                pltpu.async_copy(hs_hbm.at[src_v.at[j0 + 2]], rows0, sem0)

            pltpu.make_async_copy(hs_hbm.at[src_v.at[j1]], rows1, sem1).wait()
            pltpu.sync_copy(rows1, agg_sh.at[dst_v.at[j1]], add=True)

            @pl.when(j1 + 2 < kc)
            def _():
                pltpu.async_copy(hs_hbm.at[src_v.at[j1 + 2]], rows1, sem1)

            return c

        lax.fori_loop(0, kc // 2, step, 0)
        plsc.subcore_barrier()
        pltpu.sync_copy(agg_sh.at[pl.ds(sid * _STRIPE, _STRIPE)],
                        out_hbm.at[cid, pl.ds(sid * _STRIPE, _STRIPE)])

    return agg_kernel


def _make_pool():
    @functools.partial(
        pl.kernel,
        out_type=(jax.ShapeDtypeStruct((_NW, _TS), jnp.float32),
                  jax.ShapeDtypeStruct((_NW, _TS), jnp.float32)),
        mesh=_mesh(),
        compiler_params=pltpu.CompilerParams(use_tc_tiling_on_sc=False, needs_layout_passes=False),
        scratch_types=[
            pltpu.VMEM((_PR * _H,), jnp.float32),
            pltpu.VMEM((_PR + _L,), jnp.int32),
            pltpu.VMEM((_TS,), jnp.float32),
            pltpu.VMEM((_TS,), jnp.float32),
        ],
    )
    def pool_kernel(yflat_hbm, batch_hbm, stab0_hbm, mtab0_hbm,
                    outs_hbm, outm_hbm, y_v, b_v, stab, mtab):
        cid = lax.axis_index("c")
        sid = lax.axis_index("s")
        wid = cid * _NS + sid
        pltpu.sync_copy(yflat_hbm.at[pl.ds(wid * _PR * _H, _PR * _H)], y_v)
        pltpu.sync_copy(batch_hbm.at[pl.ds(wid * _PR, _PR + _L)], b_v)
        pltpu.sync_copy(stab0_hbm, stab)
        pltpu.sync_copy(mtab0_hbm, mtab)

        lanes = lax.iota(jnp.int32, _L)
        neg = jnp.full((_L,), -jnp.inf, jnp.float32)
        zero = jnp.zeros((_L,), jnp.float32)

        def row(i, k):
            return y_v[pl.ds(i * _H + k * _L, _L)]

        def step(i, carry):
            bprev, ms, ss = carry
            b = b_v[pl.ds(i, _L)][0]
            rs = tuple(row(i, k) for k in range(_H // _L))
            flush = b != bprev
            fm = jnp.broadcast_to(flush, (_L,))
            base = bprev * _H + lanes
            for k in range(_H // _L):
                plsc.store_scatter(stab, [base + k * _L], ss[k], mask=fm)
                plsc.store_scatter(mtab, [base + k * _L], ms[k], mask=fm)
            ms = tuple(jnp.where(fm, rs[k], jnp.maximum(ms[k], rs[k]))
                       for k in range(_H // _L))
            ss = tuple(jnp.where(fm, rs[k], ss[k] + rs[k])
                       for k in range(_H // _L))
            return b, ms, ss

        init = (b_v[pl.ds(0, _L)][0], (neg,) * (_H // _L), (zero,) * (_H // _L))
        bl, ms, ss = lax.fori_loop(0, _PR, step, init)
        base = bl * _H + lanes
        for k in range(_H // _L):
            plsc.store_scatter(stab, [base + k * _L], ss[k])
            plsc.store_scatter(mtab, [base + k * _L], ms[k])
        pltpu.sync_copy(stab, outs_hbm.at[wid])
        pltpu.sync_copy(mtab, outm_hbm.at[wid])

    return pool_kernel


def _tc_matmul1(xp, W1):
    def body(x_ref, w_ref, o_ref):
        o_ref[:] = jnp.dot(x_ref[:], w_ref[:], preferred_element_type=jnp.float32)

    return pl.pallas_call(
        body,
        grid=(_NPAD // _BR,),
        in_specs=[pl.BlockSpec((_BR, _F), lambda i: (i, 0)),
                  pl.BlockSpec((_F, _H), lambda i: (0, 0))],
        out_specs=pl.BlockSpec((_BR, _H), lambda i: (i, 0)),
        out_shape=jax.ShapeDtypeStruct((_NPAD, _H), jnp.float32),
    )(xp, W1)


def _dinv_of(degp_blk):
    deg = degp_blk[0, :, 0] + degp_blk[1, :, 0] + 1.0
    return lax.rsqrt(deg)


def _tc_hs1(degp, h1):
    def body(d_ref, h_ref, o_ref):
        dinv = _dinv_of(d_ref[:])
        o_ref[:] = h_ref[:] * dinv[:, None]

    return pl.pallas_call(
        body,
        grid=(_NPAD // _BR,),
        in_specs=[pl.BlockSpec((_NC, _BR, _DEGW), lambda i: (0, i, 0)),
                  pl.BlockSpec((_BR, _H), lambda i: (i, 0))],
        out_specs=pl.BlockSpec((_BR, _H), lambda i: (i, 0)),
        out_shape=jax.ShapeDtypeStruct((_NPAD, _H), jnp.float32),
    )(degp, h1)


def _bn_relu(t, g, be, rm, rv):
    return jax.nn.relu((t - rm) * lax.rsqrt(rv + _EPS) * g + be)


def _tc_epi1(aggp, degp, hs1, b1, g1, be1, rm1, rv1, W2):
    def body(a_ref, d_ref, h_ref, b_ref, g_ref, be_ref, rm_ref, rv_ref,
             w_ref, o_ref):
        dinv = _dinv_of(d_ref[:])
        t = (a_ref[0] + a_ref[1] + h_ref[:]) * dinv[:, None] + b_ref[:]
        y = _bn_relu(t, g_ref[:], be_ref[:], rm_ref[:], rv_ref[:])
        o_ref[:] = jnp.dot(y, w_ref[:],
                           preferred_element_type=jnp.float32) * dinv[:, None]

    vspec = pl.BlockSpec((1, _H), lambda i: (0, 0))
    return pl.pallas_call(
        body,
        grid=(_NPAD // _BR,),
        in_specs=[pl.BlockSpec((_NC, _BR, _H), lambda i: (0, i, 0)),
                  pl.BlockSpec((_NC, _BR, _DEGW), lambda i: (0, i, 0)),
                  pl.BlockSpec((_BR, _H), lambda i: (i, 0)),
                  vspec, vspec, vspec, vspec, vspec,
                  pl.BlockSpec((_H, _H), lambda i: (0, 0))],
        out_specs=pl.BlockSpec((_BR, _H), lambda i: (i, 0)),
        out_shape=jax.ShapeDtypeStruct((_NPAD, _H), jnp.float32),
    )(aggp, degp, hs1, b1, g1, be1, rm1, rv1, W2)


def _tc_epi2(aggp, degp, hs2, b2, g2, be2, rm2, rv2):
    def body(a_ref, d_ref, h_ref, b_ref, g_ref, be_ref, rm_ref, rv_ref, o_ref):
        dinv = _dinv_of(d_ref[:])
        t = (a_ref[0] + a_ref[1] + h_ref[:]) * dinv[:, None] + b_ref[:]
        o_ref[:] = _bn_relu(t, g_ref[:], be_ref[:], rm_ref[:], rv_ref[:])

    vspec = pl.BlockSpec((1, _H), lambda i: (0, 0))
    return pl.pallas_call(
        body,
        grid=(_NPAD // _BR,),
        in_specs=[pl.BlockSpec((_NC, _BR, _H), lambda i: (0, i, 0)),
                  pl.BlockSpec((_NC, _BR, _DEGW), lambda i: (0, i, 0)),
                  pl.BlockSpec((_BR, _H), lambda i: (i, 0)),
                  vspec, vspec, vspec, vspec, vspec],
        out_specs=pl.BlockSpec((_BR, _H), lambda i: (i, 0)),
        out_shape=jax.ShapeDtypeStruct((_NPAD, _H), jnp.float32),
    )(aggp, degp, hs2, b2, g2, be2, rm2, rv2)


def _tc_readout(sump, maxp, batch2d, Wr1, br1, Wr2, br2):
    def body(s_ref, m_ref, bt_ref, w1_ref, b1_ref, w2_ref, b2_ref, o_ref):
        sums = jnp.sum(s_ref[:], axis=0)[:_G]
        maxs = jnp.max(m_ref[:], axis=0)[:_G]
        seg = lax.broadcasted_iota(jnp.int32, (_G,) + bt_ref.shape, 0)
        eq = (bt_ref[:][None, :, :] == seg).astype(jnp.float32)
        cnt = jnp.sum(eq, axis=(1, 2))
        mean = sums / jnp.maximum(cnt, 1.0)[:, None]
        z = jnp.concatenate([mean, maxs], axis=1)
        r = jax.nn.relu(jnp.dot(z, w1_ref[:],
                                preferred_element_type=jnp.float32) + b1_ref[:])
        o_ref[:] = jnp.dot(r, w2_ref[:],
                           preferred_element_type=jnp.float32) + b2_ref[:]

    return pl.pallas_call(
        body,
        out_shape=jax.ShapeDtypeStruct((_G, 1), jnp.float32),
    )(sump, maxp, batch2d, Wr1, br1, Wr2, br2)


def kernel(x, edge_index, batch, W1, b1, g1, be1, rm1, rv1,
           W2, b2, g2, be2, rm2, rv2, Wr1, br1, Wr2, br2):
    E = edge_index.shape[1]
    kc = -(-E // (_NW * _CHUNK))
    kc += kc % 2
    epad = _NW * kc * _CHUNK

    src = jnp.pad(edge_index[0], (0, epad - E),
                  constant_values=_N).reshape(_NW, kc, _CHUNK)
    dst = jnp.pad(edge_index[1], (0, epad - E),
                  constant_values=_N).reshape(_NW, kc, _CHUNK)
    xp = jnp.pad(x, ((0, _NPAD - _N), (0, 0)))
    batchp = jnp.pad(batch, (0, _NPAD - _N), constant_values=_G)
    batchpl = jnp.pad(batch, (0, _NPAD + _L - _N), constant_values=_G)

    zeros_deg = jnp.zeros((_NPAD, _DEGW), jnp.float32)
    ones_deg = jnp.ones((_CHUNK, _DEGW), jnp.float32)
    zeros_h = jnp.zeros((_NPAD, _H), jnp.float32)
    stab0 = jnp.zeros((_TS,), jnp.float32)
    mtab0 = jnp.full((_TS,), -jnp.inf, jnp.float32)

    agg = _make_agg(kc)

    degp = _make_deg(kc)(dst, zeros_deg, ones_deg)
    h1 = _tc_matmul1(xp, W1)
    hs1 = _tc_hs1(degp, h1)
    aggp1 = agg(hs1, src, dst, zeros_h)
    hs2 = _tc_epi1(aggp1, degp, hs1, b1.reshape(1, _H), g1.reshape(1, _H),
                   be1.reshape(1, _H), rm1.reshape(1, _H), rv1.reshape(1, _H),
                   W2)
    aggp2 = agg(hs2, src, dst, zeros_h)
    y3 = _tc_epi2(aggp2, degp, hs2, b2.reshape(1, _H), g2.reshape(1, _H),
                  be2.reshape(1, _H), rm2.reshape(1, _H), rv2.reshape(1, _H))
    sumsf, maxsf = _make_pool()(y3.reshape(-1), batchpl, stab0, mtab0)
    out = _tc_readout(sumsf.reshape(_NW, _G + 1, _H),
                      maxsf.reshape(_NW, _G + 1, _H),
                      batchp.reshape(-1, 128), Wr1,
                      br1.reshape(1, _RH), Wr2, br2.reshape(1, 1))
    return out


# speedup vs baseline: 31.5119x; 1.9766x over previous
import functools

import jax
import jax.numpy as jnp
from jax import lax
from jax.experimental import pallas as pl
from jax.experimental.pallas import tpu as pltpu
from jax.experimental.pallas import tpu_sc as plsc

_NC, _NS, _L = 2, 16, 16
_NW = _NC * _NS

_N, _F, _H, _G = 10000, 128, 64, 64
_RH = max(_H // 2, 16)
_EPS = 1e-5
_NPAD = 10240
_CHUNK = 128
_DEGW = 16
_STRIPE = _NPAD // _NS
_PR = _NPAD // _NW
_TS = (_G + 1) * _H
_BR = 1024


def _mesh():
    return plsc.VectorSubcoreMesh(core_axis_name="c", subcore_axis_name="s")


def _make_deg(kc):
    @functools.partial(
        pl.kernel,
        out_type=jax.ShapeDtypeStruct((_NC, _NPAD, _DEGW), jnp.float32),
        mesh=_mesh(),
        compiler_params=pltpu.CompilerParams(use_tc_tiling_on_sc=False, needs_layout_passes=False),
        scratch_types=[
            pltpu.VMEM((kc, _CHUNK), jnp.int32),
            pltpu.VMEM((_CHUNK, _DEGW), jnp.float32),
            pltpu.VMEM_SHARED((_NPAD, _DEGW), jnp.float32),
        ],
    )
    def deg_kernel(dst_hbm, zeros_hbm, ones_hbm, out_hbm, dst_v, ones_v, deg_sh):
        cid = lax.axis_index("c")
        sid = lax.axis_index("s")
        wid = cid * _NS + sid
        pltpu.sync_copy(zeros_hbm.at[pl.ds(sid * _STRIPE, _STRIPE)],
                        deg_sh.at[pl.ds(sid * _STRIPE, _STRIPE)])
        pltpu.sync_copy(ones_hbm, ones_v)
        pltpu.sync_copy(dst_hbm.at[wid], dst_v)
        plsc.subcore_barrier()

        def step(j, c):
            pltpu.sync_copy(ones_v, deg_sh.at[dst_v.at[j]], add=True)
            return c

        lax.fori_loop(0, kc, step, 0)
        plsc.subcore_barrier()
        pltpu.sync_copy(deg_sh.at[pl.ds(sid * _STRIPE, _STRIPE)],
                        out_hbm.at[cid, pl.ds(sid * _STRIPE, _STRIPE)])

    return deg_kernel


def _make_agg(kc):
    @functools.partial(
        pl.kernel,
        out_type=jax.ShapeDtypeStruct((_NC, _NPAD, _H), jnp.float32),
        mesh=_mesh(),
        compiler_params=pltpu.CompilerParams(use_tc_tiling_on_sc=False, needs_layout_passes=False),
        scratch_types=[
            pltpu.VMEM((kc, _CHUNK), jnp.int32),
            pltpu.VMEM((kc, _CHUNK), jnp.int32),
            pltpu.VMEM((_CHUNK, _H), jnp.float32),
            pltpu.VMEM((_CHUNK, _H), jnp.float32),
            pltpu.VMEM_SHARED((_NPAD, _H), jnp.float32),
            pltpu.VMEM_SHARED((_NPAD, _H), jnp.float32),
            pltpu.SemaphoreType.DMA,
            pltpu.SemaphoreType.DMA,
        ],
    )
    def agg_kernel(hs_hbm, src_hbm, dst_hbm, zeros_hbm, out_hbm,
                   src_v, dst_v, rows0, rows1, agg_sh, hs_sh, sem0, sem1):
        cid = lax.axis_index("c")
        sid = lax.axis_index("s")
        wid = cid * _NS + sid
        pltpu.sync_copy(zeros_hbm.at[pl.ds(sid * _STRIPE, _STRIPE)],
                        agg_sh.at[pl.ds(sid * _STRIPE, _STRIPE)])
        pltpu.sync_copy(hs_hbm.at[pl.ds(sid * _STRIPE, _STRIPE)],
                        hs_sh.at[pl.ds(sid * _STRIPE, _STRIPE)])
        pltpu.sync_copy(src_hbm.at[wid], src_v)
        pltpu.sync_copy(dst_hbm.at[wid], dst_v)
        plsc.subcore_barrier()

        pltpu.async_copy(hs_sh.at[src_v.at[0]], rows0, sem0)
        pltpu.async_copy(hs_sh.at[src_v.at[1]], rows1, sem1)

        def step(p, c):
            j0 = 2 * p
            j1 = 2 * p + 1
            pltpu.make_async_copy(hs_sh.at[src_v.at[j0]], rows0, sem0).wait()
            pltpu.sync_copy(rows0, agg_sh.at[dst_v.at[j0]], add=True)

            @pl.when(j0 + 2 < kc)
            def _():
                pltpu.async_copy(hs_sh.at[src_v.at[j0 + 2]], rows0, sem0)

            pltpu.make_async_copy(hs_sh.at[src_v.at[j1]], rows1, sem1).wait()
            pltpu.sync_copy(rows1, agg_sh.at[dst_v.at[j1]], add=True)

            @pl.when(j1 + 2 < kc)
            def _():
                pltpu.async_copy(hs_sh.at[src_v.at[j1 + 2]], rows1, sem1)

            return c

        lax.fori_loop(0, kc // 2, step, 0)
        plsc.subcore_barrier()
        pltpu.sync_copy(agg_sh.at[pl.ds(sid * _STRIPE, _STRIPE)],
                        out_hbm.at[cid, pl.ds(sid * _STRIPE, _STRIPE)])

    return agg_kernel


def _make_pool():
    @functools.partial(
        pl.kernel,
        out_type=(jax.ShapeDtypeStruct((_NW, _TS), jnp.float32),
                  jax.ShapeDtypeStruct((_NW, _TS), jnp.float32)),
        mesh=_mesh(),
        compiler_params=pltpu.CompilerParams(use_tc_tiling_on_sc=False, needs_layout_passes=False),
        scratch_types=[
            pltpu.VMEM((_PR * _H,), jnp.float32),
            pltpu.VMEM((_PR + _L,), jnp.int32),
            pltpu.VMEM((_TS,), jnp.float32),
            pltpu.VMEM((_TS,), jnp.float32),
        ],
    )
    def pool_kernel(yflat_hbm, batch_hbm, stab0_hbm, mtab0_hbm,
                    outs_hbm, outm_hbm, y_v, b_v, stab, mtab):
        cid = lax.axis_index("c")
        sid = lax.axis_index("s")
        wid = cid * _NS + sid
        pltpu.sync_copy(yflat_hbm.at[pl.ds(wid * _PR * _H, _PR * _H)], y_v)
        pltpu.sync_copy(batch_hbm.at[pl.ds(wid * _PR, _PR + _L)], b_v)
        pltpu.sync_copy(stab0_hbm, stab)
        pltpu.sync_copy(mtab0_hbm, mtab)

        lanes = lax.iota(jnp.int32, _L)
        neg = jnp.full((_L,), -jnp.inf, jnp.float32)
        zero = jnp.zeros((_L,), jnp.float32)

        def row(i, k):
            return y_v[pl.ds(i * _H + k * _L, _L)]

        def step(i, carry):
            bprev, ms, ss = carry
            b = b_v[pl.ds(i, _L)][0]
            rs = tuple(row(i, k) for k in range(_H // _L))
            flush = b != bprev
            fm = jnp.broadcast_to(flush, (_L,))
            base = bprev * _H + lanes
            for k in range(_H // _L):
                plsc.store_scatter(stab, [base + k * _L], ss[k], mask=fm)
                plsc.store_scatter(mtab, [base + k * _L], ms[k], mask=fm)
            ms = tuple(jnp.where(fm, rs[k], jnp.maximum(ms[k], rs[k]))
                       for k in range(_H // _L))
            ss = tuple(jnp.where(fm, rs[k], ss[k] + rs[k])
                       for k in range(_H // _L))
            return b, ms, ss

        init = (b_v[pl.ds(0, _L)][0], (neg,) * (_H // _L), (zero,) * (_H // _L))
        bl, ms, ss = lax.fori_loop(0, _PR, step, init)
        base = bl * _H + lanes
        for k in range(_H // _L):
            plsc.store_scatter(stab, [base + k * _L], ss[k])
            plsc.store_scatter(mtab, [base + k * _L], ms[k])
        pltpu.sync_copy(stab, outs_hbm.at[wid])
        pltpu.sync_copy(mtab, outm_hbm.at[wid])

    return pool_kernel


def _tc_matmul1(xp, W1):
    def body(x_ref, w_ref, o_ref):
        o_ref[:] = jnp.dot(x_ref[:], w_ref[:], preferred_element_type=jnp.float32)

    return pl.pallas_call(
        body,
        grid=(_NPAD // _BR,),
        in_specs=[pl.BlockSpec((_BR, _F), lambda i: (i, 0)),
                  pl.BlockSpec((_F, _H), lambda i: (0, 0))],
        out_specs=pl.BlockSpec((_BR, _H), lambda i: (i, 0)),
        out_shape=jax.ShapeDtypeStruct((_NPAD, _H), jnp.float32),
    )(xp, W1)


def _dinv_of(degp_blk):
    deg = degp_blk[0, :, 0] + degp_blk[1, :, 0] + 1.0
    return lax.rsqrt(deg)


def _tc_hs1(degp, h1):
    def body(d_ref, h_ref, o_ref):
        dinv = _dinv_of(d_ref[:])
        o_ref[:] = h_ref[:] * dinv[:, None]

    return pl.pallas_call(
        body,
        grid=(_NPAD // _BR,),
        in_specs=[pl.BlockSpec((_NC, _BR, _DEGW), lambda i: (0, i, 0)),
                  pl.BlockSpec((_BR, _H), lambda i: (i, 0))],
        out_specs=pl.BlockSpec((_BR, _H), lambda i: (i, 0)),
        out_shape=jax.ShapeDtypeStruct((_NPAD, _H), jnp.float32),
    )(degp, h1)


def _bn_relu(t, g, be, rm, rv):
    return jax.nn.relu((t - rm) * lax.rsqrt(rv + _EPS) * g + be)


def _tc_epi1(aggp, degp, hs1, b1, g1, be1, rm1, rv1, W2):
    def body(a_ref, d_ref, h_ref, b_ref, g_ref, be_ref, rm_ref, rv_ref,
             w_ref, o_ref):
        dinv = _dinv_of(d_ref[:])
        t = (a_ref[0] + a_ref[1] + h_ref[:]) * dinv[:, None] + b_ref[:]
        y = _bn_relu(t, g_ref[:], be_ref[:], rm_ref[:], rv_ref[:])
        o_ref[:] = jnp.dot(y, w_ref[:],
                           preferred_element_type=jnp.float32) * dinv[:, None]

    vspec = pl.BlockSpec((1, _H), lambda i: (0, 0))
    return pl.pallas_call(
        body,
        grid=(_NPAD // _BR,),
        in_specs=[pl.BlockSpec((_NC, _BR, _H), lambda i: (0, i, 0)),
                  pl.BlockSpec((_NC, _BR, _DEGW), lambda i: (0, i, 0)),
                  pl.BlockSpec((_BR, _H), lambda i: (i, 0)),
                  vspec, vspec, vspec, vspec, vspec,
                  pl.BlockSpec((_H, _H), lambda i: (0, 0))],
        out_specs=pl.BlockSpec((_BR, _H), lambda i: (i, 0)),
        out_shape=jax.ShapeDtypeStruct((_NPAD, _H), jnp.float32),
    )(aggp, degp, hs1, b1, g1, be1, rm1, rv1, W2)


def _tc_epi2(aggp, degp, hs2, b2, g2, be2, rm2, rv2):
    def body(a_ref, d_ref, h_ref, b_ref, g_ref, be_ref, rm_ref, rv_ref, o_ref):
        dinv = _dinv_of(d_ref[:])
        t = (a_ref[0] + a_ref[1] + h_ref[:]) * dinv[:, None] + b_ref[:]
        o_ref[:] = _bn_relu(t, g_ref[:], be_ref[:], rm_ref[:], rv_ref[:])

    vspec = pl.BlockSpec((1, _H), lambda i: (0, 0))
    return pl.pallas_call(
        body,
        grid=(_NPAD // _BR,),
        in_specs=[pl.BlockSpec((_NC, _BR, _H), lambda i: (0, i, 0)),
                  pl.BlockSpec((_NC, _BR, _DEGW), lambda i: (0, i, 0)),
                  pl.BlockSpec((_BR, _H), lambda i: (i, 0)),
                  vspec, vspec, vspec, vspec, vspec],
        out_specs=pl.BlockSpec((_BR, _H), lambda i: (i, 0)),
        out_shape=jax.ShapeDtypeStruct((_NPAD, _H), jnp.float32),
    )(aggp, degp, hs2, b2, g2, be2, rm2, rv2)


def _tc_readout(sump, maxp, batch2d, Wr1, br1, Wr2, br2):
    def body(s_ref, m_ref, bt_ref, w1_ref, b1_ref, w2_ref, b2_ref, o_ref):
        sums = jnp.sum(s_ref[:], axis=0)[:_G]
        maxs = jnp.max(m_ref[:], axis=0)[:_G]
        seg = lax.broadcasted_iota(jnp.int32, (_G,) + bt_ref.shape, 0)
        eq = (bt_ref[:][None, :, :] == seg).astype(jnp.float32)
        cnt = jnp.sum(eq, axis=(1, 2))
        mean = sums / jnp.maximum(cnt, 1.0)[:, None]
        z = jnp.concatenate([mean, maxs], axis=1)
        r = jax.nn.relu(jnp.dot(z, w1_ref[:],
                                preferred_element_type=jnp.float32) + b1_ref[:])
        o_ref[:] = jnp.dot(r, w2_ref[:],
                           preferred_element_type=jnp.float32) + b2_ref[:]

    return pl.pallas_call(
        body,
        out_shape=jax.ShapeDtypeStruct((_G, 1), jnp.float32),
    )(sump, maxp, batch2d, Wr1, br1, Wr2, br2)


def kernel(x, edge_index, batch, W1, b1, g1, be1, rm1, rv1,
           W2, b2, g2, be2, rm2, rv2, Wr1, br1, Wr2, br2):
    E = edge_index.shape[1]
    kc = -(-E // (_NW * _CHUNK))
    kc += kc % 2
    epad = _NW * kc * _CHUNK

    src = jnp.pad(edge_index[0], (0, epad - E),
                  constant_values=_N).reshape(_NW, kc, _CHUNK)
    dst = jnp.pad(edge_index[1], (0, epad - E),
                  constant_values=_N).reshape(_NW, kc, _CHUNK)
    xp = jnp.pad(x, ((0, _NPAD - _N), (0, 0)))
    batchp = jnp.pad(batch, (0, _NPAD - _N), constant_values=_G)
    batchpl = jnp.pad(batch, (0, _NPAD + _L - _N), constant_values=_G)

    zeros_deg = jnp.zeros((_NPAD, _DEGW), jnp.float32)
    ones_deg = jnp.ones((_CHUNK, _DEGW), jnp.float32)
    zeros_h = jnp.zeros((_NPAD, _H), jnp.float32)
    stab0 = jnp.zeros((_TS,), jnp.float32)
    mtab0 = jnp.full((_TS,), -jnp.inf, jnp.float32)

    agg = _make_agg(kc)

    degp = _make_deg(kc)(dst, zeros_deg, ones_deg)
    h1 = _tc_matmul1(xp, W1)
    hs1 = _tc_hs1(degp, h1)
    aggp1 = agg(hs1, src, dst, zeros_h)
    hs2 = _tc_epi1(aggp1, degp, hs1, b1.reshape(1, _H), g1.reshape(1, _H),
                   be1.reshape(1, _H), rm1.reshape(1, _H), rv1.reshape(1, _H),
                   W2)
    aggp2 = agg(hs2, src, dst, zeros_h)
    y3 = _tc_epi2(aggp2, degp, hs2, b2.reshape(1, _H), g2.reshape(1, _H),
                  be2.reshape(1, _H), rm2.reshape(1, _H), rv2.reshape(1, _H))
    sumsf, maxsf = _make_pool()(y3.reshape(-1), batchpl, stab0, mtab0)
    out = _tc_readout(sumsf.reshape(_NW, _G + 1, _H),
                      maxsf.reshape(_NW, _G + 1, _H),
                      batchp.reshape(-1, 128), Wr1,
                      br1.reshape(1, _RH), Wr2, br2.reshape(1, 1))
    return out
